# Initial kernel scaffold; baseline (speedup 1.0000x reference)
import jax
import jax.numpy as jnp
from jax.experimental import pallas as pl


def kernel(node_feats, rel_feats, triples, obj_to_img, node_in_W, node_in_b, rel_in_W, rel_in_b, proj_W, proj_b, ln_g, ln_b, tok_W, tok_b, glob_W, glob_b):
    raise NotImplementedError("write your pallas kernel here")


# speedup vs baseline: 4.9243x; 4.9243x over previous
import functools

import jax
import jax.numpy as jnp
from jax import lax
from jax.experimental import pallas as pl
from jax.experimental.pallas import tpu as pltpu
from jax.experimental.pallas import tpu_sc as plsc

H = 128
NN = 10000
NPAD = 10240
NT = 320000
NIMG = 16
MTOK = 32
NL = 3
E2 = 2 * NT
CHW = 128
NW = 32
EPAD = 655360
EPW = EPAD // NW
NCH = EPW // CHW
RPT = NPAD // 16
NBLK = 1024
NGRID = NPAD // NBLK


def _make_segsum():
    mesh = plsc.VectorSubcoreMesh(core_axis_name="c", subcore_axis_name="s")

    @functools.partial(
        pl.kernel,
        mesh=mesh,
        out_type=jax.ShapeDtypeStruct((2, NPAD, H), jnp.float32),
        scratch_types=[
            pltpu.VMEM((1, CHW), jnp.int32),
            pltpu.VMEM((1, CHW), jnp.int32),
            pltpu.VMEM((CHW, H), jnp.float32),
            pltpu.VMEM_SHARED((NPAD, H), jnp.float32),
            pltpu.SemaphoreType.DMA,
        ],
    )
    def segsum(vals, gidx, sidx, zeros, out, gq, sq, rows, agg, sem):
        cid = lax.axis_index("c")
        sid = lax.axis_index("s")
        wid = sid * 2 + cid
        pltpu.sync_copy(zeros.at[pl.ds(sid * RPT, RPT)],
                        agg.at[pl.ds(sid * RPT, RPT)])
        plsc.subcore_barrier()

        def body(c, carry):
            pltpu.sync_copy(gidx.at[wid * NCH + c], gq)
            pltpu.sync_copy(sidx.at[wid * NCH + c], sq)
            pltpu.async_copy(vals.at[gq.at[0]], rows, sem).wait()
            pltpu.sync_copy(rows, agg.at[sq.at[0]], add=True)
            return carry

        lax.fori_loop(0, NCH, body, 0)
        plsc.subcore_barrier()
        pltpu.sync_copy(agg.at[pl.ds(sid * RPT, RPT)],
                        out.at[cid, pl.ds(sid * RPT, RPT)])

    return segsum


_segsum = _make_segsum()


def _make_rowgather():
    mesh = plsc.VectorSubcoreMesh(core_axis_name="c", subcore_axis_name="s")
    bpw = (NIMG * MTOK) // NW

    @functools.partial(
        pl.kernel,
        mesh=mesh,
        out_type=jax.ShapeDtypeStruct((NIMG * MTOK, H), jnp.float32),
        scratch_types=[
            pltpu.VMEM((bpw,), jnp.int32),
            pltpu.VMEM((bpw, H), jnp.float32),
            pltpu.SemaphoreType.DMA,
        ],
    )
    def rowgather(table, idx, out, idx_v, rows_v, sem):
        cid = lax.axis_index("c")
        sid = lax.axis_index("s")
        wid = sid * 2 + cid
        pltpu.sync_copy(idx.at[pl.ds(wid * bpw, bpw)], idx_v)
        pltpu.async_copy(table.at[idx_v], rows_v, sem).wait()
        pltpu.sync_copy(rows_v, out.at[pl.ds(wid * bpw, bpw)])

    return rowgather


_rowgather = _make_rowgather()


def _dotT(x, w):
    return lax.dot_general(x, w, (((1,), (1,)), ((), ())),
                           preferred_element_type=jnp.float32)


def _in_proj_body(nf, rf, wn, bn, wr, br, ns, rs):
    ns[...] = _dotT(nf[...], wn[...]) + bn[...]
    rs[...] = _dotT(rf[...], wr[...]) + br[...]


def _in_proj(nf, rf, wn, bn, wr, br):
    return pl.pallas_call(
        _in_proj_body,
        grid=(NGRID,),
        in_specs=[
            pl.BlockSpec((NBLK, H), lambda i: (i, 0)),
            pl.BlockSpec((NBLK, H), lambda i: (i, 0)),
            pl.BlockSpec((H, H), lambda i: (0, 0)),
            pl.BlockSpec((1, H), lambda i: (0, 0)),
            pl.BlockSpec((H, H), lambda i: (0, 0)),
            pl.BlockSpec((1, H), lambda i: (0, 0)),
        ],
        out_specs=[
            pl.BlockSpec((NBLK, H), lambda i: (i, 0)),
            pl.BlockSpec((NBLK, H), lambda i: (i, 0)),
        ],
        out_shape=[
            jax.ShapeDtypeStruct((NPAD, H), jnp.float32),
            jax.ShapeDtypeStruct((NPAD, H), jnp.float32),
        ],
    )(nf, rf, wn, bn, wr, br)


def _layer_body(ns, a, rc, w, b, out):
    agg = a[0] + a[1] + rc[0] + rc[1]
    x = ns[...]
    wf = w[...]
    h = _dotT(x, wf[:, :H]) + _dotT(agg, wf[:, H:]) + b[...]
    out[...] = x + h * jax.nn.sigmoid(h)


def _layer(ns, a, rc, w, b):
    return pl.pallas_call(
        _layer_body,
        grid=(NGRID,),
        in_specs=[
            pl.BlockSpec((NBLK, H), lambda i: (i, 0)),
            pl.BlockSpec((2, NBLK, H), lambda i: (0, i, 0)),
            pl.BlockSpec((2, NBLK, H), lambda i: (0, i, 0)),
            pl.BlockSpec((H, 2 * H), lambda i: (0, 0)),
            pl.BlockSpec((1, H), lambda i: (0, 0)),
        ],
        out_specs=pl.BlockSpec((NBLK, H), lambda i: (i, 0)),
        out_shape=jax.ShapeDtypeStruct((NPAD, H), jnp.float32),
    )(ns, a, rc, w, b)


def _final_body(ns, oim, lg, lb, gw, gb, ln, gtok, sums, cnts):
    i = pl.program_id(0)
    x = ns[...]
    m = jnp.mean(x, axis=1, keepdims=True)
    v = jnp.mean((x - m) ** 2, axis=1, keepdims=True)
    y = (x - m) * lax.rsqrt(v + 1e-5) * lg[...] + lb[...]
    ln[...] = y
    img = oim[...].reshape(1, NBLK)
    oh = (lax.broadcasted_iota(jnp.int32, (NIMG, NBLK), 0) == img
          ).astype(jnp.float32)
    bs = lax.dot_general(oh, y, (((1,), (0,)), ((), ())),
                         preferred_element_type=jnp.float32)
    bc = jnp.broadcast_to(jnp.sum(oh, axis=1, keepdims=True), (NIMG, H))

    @pl.when(i == 0)
    def _():
        sums[...] = bs
        cnts[...] = bc

    @pl.when(i > 0)
    def _():
        sums[...] += bs
        cnts[...] += bc

    @pl.when(i == NGRID - 1)
    def _():
        glob = sums[...] / jnp.maximum(cnts[...], 1.0)
        gtok[...] = _dotT(glob, gw[...]) + gb[...]


def _final(ns, oim, lg, lb, gw, gb):
    return pl.pallas_call(
        _final_body,
        grid=(NGRID,),
        in_specs=[
            pl.BlockSpec((NBLK, H), lambda i: (i, 0)),
            pl.BlockSpec((1, 1, NBLK), lambda i: (i, 0, 0)),
            pl.BlockSpec((1, H), lambda i: (0, 0)),
            pl.BlockSpec((1, H), lambda i: (0, 0)),
            pl.BlockSpec((H, H), lambda i: (0, 0)),
            pl.BlockSpec((1, H), lambda i: (0, 0)),
        ],
        out_specs=[
            pl.BlockSpec((NBLK, H), lambda i: (i, 0)),
            pl.BlockSpec((NIMG, H), lambda i: (0, 0)),
        ],
        out_shape=[
            jax.ShapeDtypeStruct((NPAD, H), jnp.float32),
            jax.ShapeDtypeStruct((NIMG, H), jnp.float32),
        ],
        scratch_shapes=[
            pltpu.VMEM((NIMG, H), jnp.float32),
            pltpu.VMEM((NIMG, H), jnp.float32),
        ],
    )(ns, oim, lg, lb, gw, gb)


def _tok_body(g, w, b, vm, out):
    out[...] = (_dotT(g[...], w[...]) + b[...]) * vm[...]


def _tok(g, w, b, vm):
    n = NIMG * MTOK
    return pl.pallas_call(
        _tok_body,
        in_specs=[
            pl.BlockSpec((n, H), lambda: (0, 0)),
            pl.BlockSpec((H, H), lambda: (0, 0)),
            pl.BlockSpec((1, H), lambda: (0, 0)),
            pl.BlockSpec((n, H), lambda: (0, 0)),
        ],
        out_specs=pl.BlockSpec((n, H), lambda: (0, 0)),
        out_shape=jax.ShapeDtypeStruct((n, H), jnp.float32),
    )(g, w, b, vm)


def kernel(node_feats, rel_feats, triples, obj_to_img, node_in_W, node_in_b,
           rel_in_W, rel_in_b, proj_W, proj_b, ln_g, ln_b, tok_W, tok_b,
           glob_W, glob_b):
    f32 = jnp.float32
    nf = jnp.pad(node_feats, ((0, NPAD - NN), (0, 0)))
    rf = jnp.pad(rel_feats, ((0, NPAD - NN), (0, 0)))

    subj = triples[:, 0].astype(jnp.int32)
    rel = triples[:, 1].astype(jnp.int32)
    obj = triples[:, 2].astype(jnp.int32)
    npadn = EPAD - E2
    ar = jnp.arange(npadn, dtype=jnp.int32)
    pad_g = ar % NN
    pad_s = NN + (ar % (NPAD - NN))
    src2 = jnp.concatenate([subj, obj, pad_g]).reshape(EPAD // CHW, 1, CHW)
    dst2 = jnp.concatenate([obj, subj, pad_s]).reshape(EPAD // CHW, 1, CHW)
    rel2 = jnp.concatenate([rel, rel, pad_g]).reshape(EPAD // CHW, 1, CHW)
    zeros = jnp.zeros((NPAD, H), f32)

    ns, rs = _in_proj(nf, rf, node_in_W, node_in_b.reshape(1, H),
                      rel_in_W, rel_in_b.reshape(1, H))
    rc = _segsum(rs, rel2, dst2, zeros)
    for l in range(NL):
        a = _segsum(ns, src2, dst2, zeros)
        ns = _layer(ns, a, rc, proj_W[l], proj_b[l].reshape(1, H))

    oim = jnp.pad(obj_to_img.astype(jnp.int32), (0, NPAD - NN),
                  constant_values=NIMG).reshape(NGRID, 1, NBLK)
    ln, gtok = _final(ns, oim, ln_g.reshape(1, H), ln_b.reshape(1, H),
                      glob_W, glob_b.reshape(1, H))

    starts17 = jnp.searchsorted(obj_to_img, jnp.arange(NIMG + 1)).astype(jnp.int32)
    starts = starts17[:NIMG]
    counts = starts17[1:] - starts
    row_idx = jnp.minimum(starts[:, None] + jnp.arange(MTOK, dtype=jnp.int32)[None, :],
                          NN - 1)
    valid = (jnp.arange(MTOK, dtype=jnp.int32)[None, :] < counts[:, None])

    g = _rowgather(ln, row_idx.reshape(NIMG * MTOK))
    vm = jnp.broadcast_to(valid.reshape(NIMG * MTOK, 1).astype(f32),
                          (NIMG * MTOK, H))
    out512 = _tok(g, tok_W, tok_b.reshape(1, H), vm)
    padded = out512.reshape(NIMG, MTOK, H)
    return jnp.concatenate([gtok[:, None, :], padded], axis=1)
